# Initial kernel scaffold; baseline (speedup 1.0000x reference)
import functools

import jax
import jax.numpy as jnp
from jax import lax
from jax.experimental import pallas as pl
from jax.experimental.pallas import tpu as pltpu
from jax.experimental.pallas import tpu_sc as plsc

_B, _T, _C = 16, 4096, 256
_OUT_LEN = 2048
_L = 16
_NC, _NS = 2, 16
_NW = _NC * _NS
_CG = _C // _L
_TASKS = _B * _CG
_TPW = _TASKS // _NW
_ROWS = _T - 1
_MAX_S = _T - _OUT_LEN - 1


def _make_sc_kernel():
    mesh = plsc.VectorSubcoreMesh(core_axis_name="c", subcore_axis_name="s")

    @functools.partial(
        pl.kernel,
        mesh=mesh,
        out_type=jax.ShapeDtypeStruct((_B, _OUT_LEN, _C), jnp.float32),
        scratch_types=[
            pltpu.VMEM((_ROWS, _L), jnp.float32),
            pltpu.VMEM((_OUT_LEN, _L), jnp.float32),
            pltpu.VMEM((_L,), jnp.float32),
        ],
    )
    def windowed_gather(signal, start_pts, out, in_v, out_v, s_v):
        wid = lax.axis_index("s") * _NC + lax.axis_index("c")
        lane = lax.iota(jnp.int32, _L)

        def task_body(i, carry):
            task = wid * _TPW + i
            b = task // _CG
            c0 = (task - b * _CG) * _L
            pltpu.sync_copy(start_pts.at[b, pl.ds(c0, _L)], s_v)
            s_i = jnp.clip(
                (s_v[...] * jnp.float32(_T - 1)).astype(jnp.int32), 0, _MAX_S
            )
            pltpu.sync_copy(signal.at[b, pl.ds(0, _ROWS), pl.ds(c0, _L)], in_v)

            def row_body(t, rows):
                out_v[t] = plsc.load_gather(in_v, [rows, lane])
                return rows + 1

            lax.fori_loop(0, _OUT_LEN, row_body, s_i)
            pltpu.sync_copy(out_v, out.at[b, pl.ds(0, _OUT_LEN), pl.ds(c0, _L)])
            return carry

        lax.fori_loop(0, _TPW, task_body, 0)

    return windowed_gather


_sc_window = _make_sc_kernel()


def kernel(signal, start_pts):
    return _sc_window(signal, start_pts)


# speedup vs baseline: 3.3328x; 3.3328x over previous
import functools

import jax
import jax.numpy as jnp
from jax import lax
from jax.experimental import pallas as pl
from jax.experimental.pallas import tpu as pltpu
from jax.experimental.pallas import tpu_sc as plsc

_B, _T, _C = 16, 4096, 256
_OUT_LEN = 2048
_L = 16
_NC, _NS = 2, 16
_NW = _NC * _NS
_CG = _C // _L
_TASKS = _B * _CG
_TPW = _TASKS // _NW
_ROWS = _T - 1
_MAX_S = _T - _OUT_LEN - 1


def _make_sc_kernel():
    mesh = plsc.VectorSubcoreMesh(core_axis_name="c", subcore_axis_name="s")

    @functools.partial(
        pl.kernel,
        mesh=mesh,
        out_type=jax.ShapeDtypeStruct((_B, _OUT_LEN, _C), jnp.float32),
        scratch_types=[
            pltpu.VMEM((_ROWS, _L), jnp.float32),
            pltpu.VMEM((_OUT_LEN, _L), jnp.float32),
            pltpu.VMEM((_L,), jnp.float32),
        ],
        compiler_params=pltpu.CompilerParams(
            use_tc_tiling_on_sc=False, needs_layout_passes=False
        ),
    )
    def windowed_gather(signal, start_pts, out, in_v, out_v, s_v):
        wid = lax.axis_index("s") * _NC + lax.axis_index("c")
        lane = lax.iota(jnp.int32, _L)

        def task_body(i, carry):
            task = wid * _TPW + i
            b = task // _CG
            c0 = (task - b * _CG) * _L
            pltpu.sync_copy(start_pts.at[b, pl.ds(c0, _L)], s_v)
            s_i = jnp.clip(
                (s_v[...] * jnp.float32(_T - 1)).astype(jnp.int32), 0, _MAX_S
            )
            pltpu.sync_copy(signal.at[b, pl.ds(0, _ROWS), pl.ds(c0, _L)], in_v)

            def row_body(t, rows):
                out_v[t] = plsc.load_gather(in_v, [rows, lane])
                return rows + 1

            lax.fori_loop(0, _OUT_LEN, row_body, s_i)
            pltpu.sync_copy(out_v, out.at[b, pl.ds(0, _OUT_LEN), pl.ds(c0, _L)])
            return carry

        lax.fori_loop(0, _TPW, task_body, 0)

    return windowed_gather


_sc_window = _make_sc_kernel()


def kernel(signal, start_pts):
    return _sc_window(signal, start_pts)


# speedup vs baseline: 4.3520x; 1.3058x over previous
import functools

import jax
import jax.numpy as jnp
from jax import lax
from jax.experimental import pallas as pl
from jax.experimental.pallas import tpu as pltpu
from jax.experimental.pallas import tpu_sc as plsc

_B, _T, _C = 16, 4096, 256
_OUT_LEN = 2048
_L = 16
_NC, _NS = 2, 16
_NW = _NC * _NS
_CG = _C // _L
_TASKS = _B * _CG
_TPW = _TASKS // _NW
_ROWS = _T - 1
_MAX_S = _T - _OUT_LEN - 1


def _make_sc_kernel():
    mesh = plsc.VectorSubcoreMesh(core_axis_name="c", subcore_axis_name="s")

    @functools.partial(
        pl.kernel,
        mesh=mesh,
        out_type=jax.ShapeDtypeStruct((_B, _OUT_LEN, _C), jnp.float32),
        scratch_types=[
            pltpu.VMEM((_ROWS, _L), jnp.float32),
            pltpu.VMEM((_OUT_LEN, _L), jnp.float32),
            pltpu.VMEM((_L,), jnp.float32),
        ],
        compiler_params=pltpu.CompilerParams(
            use_tc_tiling_on_sc=False, needs_layout_passes=False
        ),
    )
    def windowed_gather(signal, start_pts, out, in_v, out_v, s_v):
        wid = lax.axis_index("s") * _NC + lax.axis_index("c")
        lane = lax.iota(jnp.int32, _L)

        def task_body(i, carry):
            task = wid * _TPW + i
            b = task // _CG
            c0 = (task - b * _CG) * _L
            pltpu.sync_copy(start_pts.at[b, pl.ds(c0, _L)], s_v)
            s_i = jnp.clip(
                (s_v[...] * jnp.float32(_T - 1)).astype(jnp.int32), 0, _MAX_S
            )
            pltpu.sync_copy(signal.at[b, pl.ds(0, _ROWS), pl.ds(c0, _L)], in_v)

            @plsc.parallel_loop(0, _OUT_LEN, unroll=16)
            def row_body(t):
                out_v[t] = plsc.load_gather(in_v, [s_i + t, lane])
            pltpu.sync_copy(out_v, out.at[b, pl.ds(0, _OUT_LEN), pl.ds(c0, _L)])
            return carry

        lax.fori_loop(0, _TPW, task_body, 0)

    return windowed_gather


_sc_window = _make_sc_kernel()


def kernel(signal, start_pts):
    return _sc_window(signal, start_pts)


# speedup vs baseline: 8.2951x; 1.9061x over previous
import functools

import jax
import jax.numpy as jnp
from jax import lax
from jax.experimental import pallas as pl
from jax.experimental.pallas import tpu as pltpu
from jax.experimental.pallas import tpu_sc as plsc

_B, _T, _C = 16, 4096, 256
_OUT_LEN = 2048
_L = 16
_NC, _NS = 2, 16
_NW = _NC * _NS
_CG = _C // _L
_TASKS = _B * _CG
_TPW = _TASKS // _NW
_MAX_S = _T - _OUT_LEN - 1
_TT = _T // 8
_OT = _OUT_LEN // 8
_CT = _C // 128


def _make_sc_kernel():
    mesh = plsc.VectorSubcoreMesh(core_axis_name="c", subcore_axis_name="s")

    @functools.partial(
        pl.kernel,
        mesh=mesh,
        out_type=jax.ShapeDtypeStruct((_B, _OT, _CT, 8, 128), jnp.float32),
        scratch_types=[
            pltpu.VMEM((_TT, 8, _L), jnp.float32),
            pltpu.VMEM((_OT, 8, _L), jnp.float32),
            pltpu.VMEM((_L,), jnp.float32),
        ],
        compiler_params=pltpu.CompilerParams(
            use_tc_tiling_on_sc=False, needs_layout_passes=False
        ),
    )
    def windowed_gather(signal, start_pts, out, in_v, out_v, s_v):
        wid = lax.axis_index("s") * _NC + lax.axis_index("c")
        lane = lax.iota(jnp.int32, _L)

        def task_body(i, carry):
            task = wid * _TPW + i
            b = task // _CG
            g = task - b * _CG
            ct = g // 8
            l0 = (g - ct * 8) * _L
            pltpu.sync_copy(
                start_pts.at[b // 8, ct, b - (b // 8) * 8, pl.ds(l0, _L)], s_v
            )
            s_i = jnp.clip(
                (s_v[...] * jnp.float32(_T - 1)).astype(jnp.int32), 0, _MAX_S
            )
            pltpu.sync_copy(
                signal.at[b, pl.ds(0, _TT), ct, pl.ds(0, 8), pl.ds(l0, _L)],
                in_v,
            )

            for r in range(8):
                sr = s_i + r
                tt0 = sr >> 3
                rr = sr & 7

                @plsc.parallel_loop(0, _OT, unroll=16)
                def row_body(k):
                    out_v[k, r] = plsc.load_gather(in_v, [tt0 + k, rr, lane])

            pltpu.sync_copy(
                out_v, out.at[b, pl.ds(0, _OT), ct, pl.ds(0, 8), pl.ds(l0, _L)]
            )
            return carry

        lax.fori_loop(0, _TPW, task_body, 0)

    return windowed_gather


_sc_window = _make_sc_kernel()


def kernel(signal, start_pts):
    sig5 = signal.reshape(_B, _TT, 8, _CT, 128).transpose(0, 1, 3, 2, 4)
    sp4 = start_pts.reshape(_B // 8, 8, _CT, 128).transpose(0, 2, 1, 3)
    out5 = _sc_window(sig5, sp4)
    return out5.transpose(0, 1, 3, 2, 4).reshape(_B, _OUT_LEN, _C)


# speedup vs baseline: 8.5082x; 1.0257x over previous
import functools

import jax
import jax.numpy as jnp
from jax import lax
from jax.experimental import pallas as pl
from jax.experimental.pallas import tpu as pltpu
from jax.experimental.pallas import tpu_sc as plsc

_B, _T, _C = 16, 4096, 256
_OUT_LEN = 2048
_L = 16
_NC, _NS = 2, 16
_NW = _NC * _NS
_CG = _C // _L
_TASKS = _B * _CG
_TPW = _TASKS // _NW
_MAX_S = _T - _OUT_LEN - 1
_TT = _T // 8
_OT = _OUT_LEN // 8
_CT = _C // 128


def _make_sc_kernel():
    mesh = plsc.VectorSubcoreMesh(core_axis_name="c", subcore_axis_name="s")

    @functools.partial(
        pl.kernel,
        mesh=mesh,
        out_type=jax.ShapeDtypeStruct((_B, _OT, _CT, 8, 128), jnp.float32),
        scratch_types=[
            pltpu.VMEM((_TT, 8, _L), jnp.float32),
            pltpu.VMEM((_OT, 8, _L), jnp.float32),
            pltpu.VMEM((_L,), jnp.float32),
            pltpu.SemaphoreType.DMA,
            pltpu.SemaphoreType.DMA,
            pltpu.SemaphoreType.DMA,
            pltpu.SemaphoreType.DMA,
            pltpu.SemaphoreType.DMA,
        ],
        compiler_params=pltpu.CompilerParams(
            use_tc_tiling_on_sc=False, needs_layout_passes=False
        ),
    )
    def windowed_gather(signal, start_pts, out, in_v, out_v, s_v, *sems):
        *in_sems, out_sem = sems
        wid = lax.axis_index("s") * _NC + lax.axis_index("c")
        lane = lax.iota(jnp.int32, _L)

        bounds = (0, 288, 368, 448, _TT)

        def out_dst(task):
            b = task // _CG
            g = task - b * _CG
            ct = g // 8
            l0 = (g - ct * 8) * _L
            return out.at[b, pl.ds(0, _OT), ct, pl.ds(0, 8), pl.ds(l0, _L)]

        def task_body(i, carry):
            task = wid * _TPW + i
            b = task // _CG
            g = task - b * _CG
            ct = g // 8
            l0 = (g - ct * 8) * _L
            pltpu.sync_copy(
                start_pts.at[b // 8, ct, b - (b // 8) * 8, pl.ds(l0, _L)], s_v
            )
            s_i = jnp.clip(
                (s_v[...] * jnp.float32(_T - 1)).astype(jnp.int32), 0, _MAX_S
            )
            handles = [
                pltpu.async_copy(
                    signal.at[
                        b, pl.ds(lo, hi - lo), ct, pl.ds(0, 8), pl.ds(l0, _L)
                    ],
                    in_v.at[pl.ds(lo, hi - lo)],
                    sem,
                )
                for lo, hi, sem in zip(bounds[:-1], bounds[1:], in_sems)
            ]

            @pl.when(i > 0)
            def _():
                pltpu.make_async_copy(out_v, out_dst(task), out_sem).wait()

            for c, handle in enumerate(handles):
                handle.wait()
                k_lo = max(bounds[c] - 257, 0) + 1 if c else 0
                k_hi = min(bounds[c + 1] - 257 + 1, _OT)
                for r in range(8):
                    sr = s_i + r
                    tt0 = sr >> 3
                    rr = sr & 7

                    @plsc.parallel_loop(k_lo, k_hi, unroll=8)
                    def row_body(k):
                        out_v[k, r] = plsc.load_gather(
                            in_v, [tt0 + k, rr, lane]
                        )

            pltpu.async_copy(out_v, out_dst(task), out_sem)
            return carry

        lax.fori_loop(0, _TPW, task_body, 0)
        pltpu.make_async_copy(
            out_v, out_dst(wid * _TPW + _TPW - 1), out_sem
        ).wait()

    return windowed_gather


_sc_window = _make_sc_kernel()


def kernel(signal, start_pts):
    sig5 = signal.reshape(_B, _TT, 8, _CT, 128).transpose(0, 1, 3, 2, 4)
    sp4 = start_pts.reshape(_B // 8, 8, _CT, 128).transpose(0, 2, 1, 3)
    out5 = _sc_window(sig5, sp4)
    return out5.transpose(0, 1, 3, 2, 4).reshape(_B, _OUT_LEN, _C)


# speedup vs baseline: 9.5805x; 1.1260x over previous
import functools

import jax
import jax.numpy as jnp
from jax import lax
from jax.experimental import pallas as pl
from jax.experimental.pallas import tpu as pltpu
from jax.experimental.pallas import tpu_sc as plsc

_B, _T, _C = 16, 4096, 256
_OUT_LEN = 2048
_L = 16
_NC, _NS = 2, 16
_NW = _NC * _NS
_CG = _C // _L
_MAX_S = _T - _OUT_LEN - 1
_TT = _T // 8
_OT = _OUT_LEN // 8
_CT = _C // 128
_NB_SC = 6
_TASKS = _NB_SC * _CG
_TPW = _TASKS // _NW


def _make_sc_kernel():
    mesh = plsc.VectorSubcoreMesh(core_axis_name="c", subcore_axis_name="s")

    @functools.partial(
        pl.kernel,
        mesh=mesh,
        out_type=jax.ShapeDtypeStruct((_NB_SC, _OT, _CT, 8, 128), jnp.float32),
        scratch_types=[
            pltpu.VMEM((_TT, 8, _L), jnp.float32),
            pltpu.VMEM((_OT, 8, _L), jnp.float32),
            pltpu.VMEM((_L,), jnp.float32),
            pltpu.SemaphoreType.DMA,
            pltpu.SemaphoreType.DMA,
            pltpu.SemaphoreType.DMA,
            pltpu.SemaphoreType.DMA,
            pltpu.SemaphoreType.DMA,
        ],
        compiler_params=pltpu.CompilerParams(
            use_tc_tiling_on_sc=False, needs_layout_passes=False
        ),
    )
    def windowed_gather(signal, start_pts, out, in_v, out_v, s_v, *sems):
        *in_sems, out_sem = sems
        wid = lax.axis_index("s") * _NC + lax.axis_index("c")
        lane = lax.iota(jnp.int32, _L)

        bounds = (0, 288, 368, 448, _TT)

        def out_dst(task):
            b = task // _CG
            g = task - b * _CG
            ct = g // 8
            l0 = (g - ct * 8) * _L
            return out.at[b, pl.ds(0, _OT), ct, pl.ds(0, 8), pl.ds(l0, _L)]

        def task_body(i, carry):
            task = wid * _TPW + i
            b = task // _CG
            g = task - b * _CG
            ct = g // 8
            l0 = (g - ct * 8) * _L
            pltpu.sync_copy(
                start_pts.at[b // 8, ct, b - (b // 8) * 8, pl.ds(l0, _L)], s_v
            )
            s_i = jnp.clip(
                (s_v[...] * jnp.float32(_T - 1)).astype(jnp.int32), 0, _MAX_S
            )
            handles = [
                pltpu.async_copy(
                    signal.at[
                        b, pl.ds(lo, hi - lo), ct, pl.ds(0, 8), pl.ds(l0, _L)
                    ],
                    in_v.at[pl.ds(lo, hi - lo)],
                    sem,
                )
                for lo, hi, sem in zip(bounds[:-1], bounds[1:], in_sems)
            ]

            @pl.when(i > 0)
            def _():
                pltpu.make_async_copy(out_v, out_dst(task), out_sem).wait()

            for c, handle in enumerate(handles):
                handle.wait()
                k_lo = max(bounds[c] - 257, 0) + 1 if c else 0
                k_hi = min(bounds[c + 1] - 257 + 1, _OT)
                for r in range(8):
                    sr = s_i + r
                    tt0 = sr >> 3
                    rr = sr & 7

                    @plsc.parallel_loop(k_lo, k_hi, unroll=8)
                    def row_body(k):
                        out_v[k, r] = plsc.load_gather(
                            in_v, [tt0 + k, rr, lane]
                        )

            pltpu.async_copy(out_v, out_dst(task), out_sem)
            return carry

        lax.fori_loop(0, _TPW, task_body, 0)
        pltpu.make_async_copy(
            out_v, out_dst(wid * _TPW + _TPW - 1), out_sem
        ).wait()

    return windowed_gather


_sc_window = _make_sc_kernel()


def _tc_body(sp_ref, sig_ref, out_ref):
    s = jnp.clip(
        (sp_ref[0] * jnp.float32(_T - 1)).astype(jnp.int32), 0, _MAX_S
    )
    x = sig_ref[0]
    for j in range(10, -1, -1):
        length = _OUT_LEN + (1 << j) - 1
        cond = ((s >> j) & 1) == 1
        x = jnp.where(cond, x[(1 << j) : (1 << j) + length], x[:length])
    out_ref[0] = x


_tc_shift = pl.pallas_call(
    _tc_body,
    grid=(_B - _NB_SC,),
    in_specs=[
        pl.BlockSpec((1, 1, _C), lambda b: (b + _NB_SC, 0, 0)),
        pl.BlockSpec((1, _T, _C), lambda b: (b + _NB_SC, 0, 0)),
    ],
    out_specs=pl.BlockSpec((1, _OUT_LEN, _C), lambda b: (b, 0, 0)),
    out_shape=jax.ShapeDtypeStruct((_B - _NB_SC, _OUT_LEN, _C), jnp.float32),
)


def kernel(signal, start_pts):
    sig5 = signal.reshape(_B, _TT, 8, _CT, 128).transpose(0, 1, 3, 2, 4)
    sp4 = start_pts.reshape(_B // 8, 8, _CT, 128).transpose(0, 2, 1, 3)
    sc5 = _sc_window(sig5, sp4)
    sc_part = sc5.transpose(0, 1, 3, 2, 4).reshape(_NB_SC, _OUT_LEN, _C)
    tc_part = _tc_shift(start_pts.reshape(_B, 1, _C), signal)
    return jnp.concatenate([sc_part, tc_part], axis=0)


# speedup vs baseline: 11.2186x; 1.1710x over previous
import functools

import jax
import jax.numpy as jnp
from jax import lax
from jax.experimental import pallas as pl
from jax.experimental.pallas import tpu as pltpu
from jax.experimental.pallas import tpu_sc as plsc

_B, _T, _C = 16, 4096, 256
_OUT_LEN = 2048
_L = 16
_NC, _NS = 2, 16
_NW = _NC * _NS
_CG = _C // _L
_MAX_S = _T - _OUT_LEN - 1
_TT = _T // 8
_OT = _OUT_LEN // 8
_CT = _C // 128
_NB_SC = 6
_TASKS = _NB_SC * _CG
_TPW = _TASKS // _NW


def _make_sc_kernel():
    mesh = plsc.VectorSubcoreMesh(core_axis_name="c", subcore_axis_name="s")

    @functools.partial(
        pl.kernel,
        mesh=mesh,
        out_type=jax.ShapeDtypeStruct((_NB_SC, _OT, _CT, 8, 128), jnp.float32),
        scratch_types=[
            pltpu.VMEM((_TT, 8, _L), jnp.float32),
            pltpu.VMEM((_OT, 8, _L), jnp.float32),
            pltpu.VMEM((_L,), jnp.float32),
            pltpu.SemaphoreType.DMA,
            pltpu.SemaphoreType.DMA,
            pltpu.SemaphoreType.DMA,
            pltpu.SemaphoreType.DMA,
            pltpu.SemaphoreType.DMA,
        ],
        compiler_params=pltpu.CompilerParams(
            use_tc_tiling_on_sc=False, needs_layout_passes=False
        ),
    )
    def windowed_gather(signal, start_pts, out, in_v, out_v, s_v, *sems):
        *in_sems, out_sem = sems
        wid = lax.axis_index("s") * _NC + lax.axis_index("c")
        lane = lax.iota(jnp.int32, _L)

        bounds = (0, 288, 368, 448, _TT)

        def out_dst(task):
            b = task // _CG
            g = task - b * _CG
            ct = g // 8
            l0 = (g - ct * 8) * _L
            return out.at[b, pl.ds(0, _OT), ct, pl.ds(0, 8), pl.ds(l0, _L)]

        def task_body(i, carry):
            task = wid * _TPW + i
            b = task // _CG
            g = task - b * _CG
            ct = g // 8
            l0 = (g - ct * 8) * _L
            pltpu.sync_copy(
                start_pts.at[b // 8, ct, b - (b // 8) * 8, pl.ds(l0, _L)], s_v
            )
            s_i = jnp.clip(
                (s_v[...] * jnp.float32(_T - 1)).astype(jnp.int32), 0, _MAX_S
            )
            handles = [
                pltpu.async_copy(
                    signal.at[
                        b, pl.ds(lo, hi - lo), ct, pl.ds(0, 8), pl.ds(l0, _L)
                    ],
                    in_v.at[pl.ds(lo, hi - lo)],
                    sem,
                )
                for lo, hi, sem in zip(bounds[:-1], bounds[1:], in_sems)
            ]

            @pl.when(i > 0)
            def _():
                pltpu.make_async_copy(out_v, out_dst(task), out_sem).wait()

            for c, handle in enumerate(handles):
                handle.wait()
                k_lo = max(bounds[c] - 257, 0) + 1 if c else 0
                k_hi = min(bounds[c + 1] - 257 + 1, _OT)
                for r in range(8):
                    sr = s_i + r
                    tt0 = sr >> 3
                    rr = sr & 7

                    @plsc.parallel_loop(k_lo, k_hi, unroll=8)
                    def row_body(k):
                        out_v[k, r] = plsc.load_gather(
                            in_v, [tt0 + k, rr, lane]
                        )

            pltpu.async_copy(out_v, out_dst(task), out_sem)
            return carry

        lax.fori_loop(0, _TPW, task_body, 0)
        pltpu.make_async_copy(
            out_v, out_dst(wid * _TPW + _TPW - 1), out_sem
        ).wait()

    return windowed_gather


_sc_window = _make_sc_kernel()


def _tc_body(sp_ref, sig_ref, out_ref):
    s = jnp.clip(
        (sp_ref[0] * jnp.float32(_T - 1)).astype(jnp.int32), 0, _MAX_S
    )
    x = sig_ref[0]
    for j in range(10, -1, -1):
        length = _OUT_LEN + (1 << j) - 1
        cond = ((s >> j) & 1) == 1
        x = jnp.where(cond, x[(1 << j) : (1 << j) + length], x[:length])
    out_ref[0] = x


_tc_shift = pl.pallas_call(
    _tc_body,
    grid=(_B - _NB_SC,),
    in_specs=[
        pl.BlockSpec((1, 1, _C), lambda b: (b + _NB_SC, 0, 0)),
        pl.BlockSpec((1, _T, _C), lambda b: (b + _NB_SC, 0, 0)),
    ],
    out_specs=pl.BlockSpec((1, _OUT_LEN, _C), lambda b: (b + _NB_SC, 0, 0)),
    out_shape=jax.ShapeDtypeStruct((_B, _OUT_LEN, _C), jnp.float32),
)


def kernel(signal, start_pts):
    sig5 = signal.reshape(_B, _TT, 8, _CT, 128).transpose(0, 1, 3, 2, 4)
    sp4 = start_pts.reshape(_B // 8, 8, _CT, 128).transpose(0, 2, 1, 3)
    sc5 = _sc_window(sig5, sp4)
    sc_part = sc5.transpose(0, 1, 3, 2, 4).reshape(_NB_SC, _OUT_LEN, _C)
    tc_full = _tc_shift(start_pts.reshape(_B, 1, _C), signal)
    return lax.dynamic_update_slice(tc_full, sc_part, (0, 0, 0))


# speedup vs baseline: 12.1793x; 1.0856x over previous
import functools

import jax
import jax.numpy as jnp
from jax import lax
from jax.experimental import pallas as pl
from jax.experimental.pallas import tpu as pltpu
from jax.experimental.pallas import tpu_sc as plsc

_B, _T, _C = 16, 4096, 256
_OUT_LEN = 2048
_L = 16
_NC, _NS = 2, 16
_NW = _NC * _NS
_CG = _C // _L
_MAX_S = _T - _OUT_LEN - 1
_TT = _T // 8
_OT = _OUT_LEN // 8
_CT = _C // 128
_NB_SC = 4
_TASKS = _NB_SC * _CG
_TPW = _TASKS // _NW


def _make_sc_kernel():
    mesh = plsc.VectorSubcoreMesh(core_axis_name="c", subcore_axis_name="s")

    @functools.partial(
        pl.kernel,
        mesh=mesh,
        out_type=jax.ShapeDtypeStruct((_NB_SC, _OT, _CT, 8, 128), jnp.float32),
        scratch_types=[
            pltpu.VMEM((_TT, 8, _L), jnp.float32),
            pltpu.VMEM((_OT, 8, _L), jnp.float32),
            pltpu.VMEM((_L,), jnp.float32),
            pltpu.SemaphoreType.DMA,
            pltpu.SemaphoreType.DMA,
            pltpu.SemaphoreType.DMA,
            pltpu.SemaphoreType.DMA,
            pltpu.SemaphoreType.DMA,
        ],
        compiler_params=pltpu.CompilerParams(
            use_tc_tiling_on_sc=False, needs_layout_passes=False
        ),
    )
    def windowed_gather(signal, start_pts, out, in_v, out_v, s_v, *sems):
        *in_sems, out_sem = sems
        wid = lax.axis_index("s") * _NC + lax.axis_index("c")
        lane = lax.iota(jnp.int32, _L)

        bounds = (0, 288, 368, 448, _TT)

        def out_dst(task):
            b = task // _CG
            g = task - b * _CG
            ct = g // 8
            l0 = (g - ct * 8) * _L
            return out.at[b, pl.ds(0, _OT), ct, pl.ds(0, 8), pl.ds(l0, _L)]

        def task_body(i, carry):
            task = wid * _TPW + i
            b = task // _CG
            g = task - b * _CG
            ct = g // 8
            l0 = (g - ct * 8) * _L
            pltpu.sync_copy(
                start_pts.at[b // 8, ct, b - (b // 8) * 8, pl.ds(l0, _L)], s_v
            )
            s_i = jnp.clip(
                (s_v[...] * jnp.float32(_T - 1)).astype(jnp.int32), 0, _MAX_S
            )
            handles = [
                pltpu.async_copy(
                    signal.at[
                        b, pl.ds(lo, hi - lo), ct, pl.ds(0, 8), pl.ds(l0, _L)
                    ],
                    in_v.at[pl.ds(lo, hi - lo)],
                    sem,
                )
                for lo, hi, sem in zip(bounds[:-1], bounds[1:], in_sems)
            ]

            @pl.when(i > 0)
            def _():
                pltpu.make_async_copy(out_v, out_dst(task), out_sem).wait()

            for c, handle in enumerate(handles):
                handle.wait()
                k_lo = max(bounds[c] - 257, 0) + 1 if c else 0
                k_hi = min(bounds[c + 1] - 257 + 1, _OT)
                for r in range(8):
                    sr = s_i + r
                    tt0 = sr >> 3
                    rr = sr & 7

                    @plsc.parallel_loop(k_lo, k_hi, unroll=8)
                    def row_body(k):
                        out_v[k, r] = plsc.load_gather(
                            in_v, [tt0 + k, rr, lane]
                        )

            pltpu.async_copy(out_v, out_dst(task), out_sem)
            return carry

        lax.fori_loop(0, _TPW, task_body, 0)
        pltpu.make_async_copy(
            out_v, out_dst(wid * _TPW + _TPW - 1), out_sem
        ).wait()

    return windowed_gather


_sc_window = _make_sc_kernel()


_TC_CB = 128
_TC_NCB = _C // _TC_CB


def _tc_body(sp_ref, sig_ref, out_ref):
    s = jnp.clip(
        (sp_ref[0] * jnp.float32(_T - 1)).astype(jnp.int32), 0, _MAX_S
    )
    x = sig_ref[0]
    for j in range(10, -1, -1):
        length = _OUT_LEN + (1 << j) - 1
        cond = ((s >> j) & 1) == 1
        x = jnp.where(cond, x[(1 << j) : (1 << j) + length], x[:length])
    out_ref[0] = x


_tc_shift = pl.pallas_call(
    _tc_body,
    grid=(_B - _NB_SC, _TC_NCB),
    in_specs=[
        pl.BlockSpec((1, 1, _TC_CB), lambda b, c: (b + _NB_SC, 0, c)),
        pl.BlockSpec((1, _T, _TC_CB), lambda b, c: (b + _NB_SC, 0, c)),
    ],
    out_specs=pl.BlockSpec(
        (1, _OUT_LEN, _TC_CB), lambda b, c: (b + _NB_SC, 0, c)
    ),
    out_shape=jax.ShapeDtypeStruct((_B, _OUT_LEN, _C), jnp.float32),
)


def kernel(signal, start_pts):
    sig5 = signal.reshape(_B, _TT, 8, _CT, 128).transpose(0, 1, 3, 2, 4)
    sp4 = start_pts.reshape(_B // 8, 8, _CT, 128).transpose(0, 2, 1, 3)
    sc5 = _sc_window(sig5, sp4)
    sc_part = sc5.transpose(0, 1, 3, 2, 4).reshape(_NB_SC, _OUT_LEN, _C)
    tc_full = _tc_shift(start_pts.reshape(_B, 1, _C), signal)
    return lax.dynamic_update_slice(tc_full, sc_part, (0, 0, 0))
